# baseline (speedup 1.0000x reference)
import functools

import jax
import jax.numpy as jnp
from jax import lax
from jax.experimental import pallas as pl
from jax.experimental.pallas import tpu as pltpu
from jax.experimental.pallas import tpu_sc as plsc

BATCH = 16384
FIELDS = 26
OUT = 32
TOTAL = BATCH * FIELDS
NUM_CORES = 2
NUM_SUBCORES = 16
NW = NUM_CORES * NUM_SUBCORES
PER_W = TOTAL // NW
CHUNK = 128
N_CHUNKS = PER_W // CHUNK
GROUPS = CHUNK // 16
DEPTH = 8
assert PER_W * NW == TOTAL and N_CHUNKS * CHUNK == PER_W
assert N_CHUNKS % DEPTH == 0 and DEPTH % 2 == 0


def _rsqrt(x):
    i = lax.bitcast_convert_type(x, jnp.int32)
    i = jnp.full((16,), 0x5F3759DF, jnp.int32) - (i >> 1)
    y = lax.bitcast_convert_type(i, jnp.float32)
    for _ in range(3):
        y = y * (1.5 - 0.5 * x * y * y)
    return y


_mesh = plsc.VectorSubcoreMesh(core_axis_name="c", subcore_axis_name="s")


@functools.partial(
    pl.kernel,
    out_type=jax.ShapeDtypeStruct((TOTAL, OUT), jnp.float32),
    mesh=_mesh,
    scratch_types=[
        pltpu.VMEM((N_CHUNKS, CHUNK), jnp.int32),
        [pltpu.VMEM((CHUNK, OUT), jnp.float32) for _ in range(DEPTH)],
        [pltpu.VMEM((CHUNK, OUT), jnp.float32) for _ in range(2)],
        [pltpu.SemaphoreType.DMA for _ in range(DEPTH)],
        [pltpu.SemaphoreType.DMA for _ in range(2)],
    ],
    compiler_params=pltpu.CompilerParams(
        needs_layout_passes=False, use_tc_tiling_on_sc=False
    ),
)
def _gather_normalize(table_hbm, idx_hbm, out_hbm,
                      idx_v, gbufs, obufs, gsems, wsems):
    wid = lax.axis_index("s") * NUM_CORES + lax.axis_index("c")
    base = wid * PER_W

    lanes = lax.iota(jnp.int32, 16)

    def start_gather(g, p):
        pltpu.async_copy(table_hbm.at[idx_v.at[g]], gbufs[p], gsems[p])

    def wait_gather(g, p):
        pltpu.make_async_copy(table_hbm.at[idx_v.at[g]], gbufs[p],
                              gsems[p]).wait()

    def start_write(g, q):
        pltpu.async_copy(obufs[q], out_hbm.at[pl.ds(base + g * CHUNK, CHUNK)],
                         wsems[q])

    def wait_write(g, q):
        pltpu.make_async_copy(obufs[q],
                              out_hbm.at[pl.ds(base + g * CHUNK, CHUNK)],
                              wsems[q]).wait()

    def normalize(p, q):
        src_v, dst_v = gbufs[p], obufs[q]

        def group_body(gr, c):
            row_ids = gr * 16 + lanes
            cols = [(lanes + j) & 31 for j in range(OUT)]
            elems = [plsc.load_gather(src_v, [row_ids, cols[j]])
                     for j in range(OUT)]
            acc = jnp.zeros((16,), jnp.float32)
            for e in elems:
                acc = acc + e * e
            inv = _rsqrt(acc)
            for j, e in enumerate(elems):
                plsc.store_scatter(dst_v, [row_ids, cols[j]], e * inv)
            return c

        lax.fori_loop(0, GROUPS, group_body, 0)

    pltpu.sync_copy(idx_hbm.at[wid], idx_v)
    for p in range(DEPTH):
        start_gather(p, p)

    def round_body(t, carry):
        for p in range(DEPTH):
            g = t * DEPTH + p
            q = p % 2
            wait_gather(g, p)

            @pl.when(g >= 2)
            def _():
                wait_write(g - 2, q)

            normalize(p, q)
            start_write(g, q)

            @pl.when(g + DEPTH < N_CHUNKS)
            def _():
                start_gather(g + DEPTH, p)

        return carry

    lax.fori_loop(0, N_CHUNKS // DEPTH, round_body, 0)
    wait_write(N_CHUNKS - 2, 0)
    wait_write(N_CHUNKS - 1, 1)


def kernel(src, categories_means, categories_logvars):
    del categories_logvars
    idx = src.astype(jnp.int32).reshape(NW, N_CHUNKS, CHUNK)
    flat = _gather_normalize(categories_means, idx)
    return flat.reshape(BATCH, FIELDS, OUT)


# speedup vs baseline: 1.1326x; 1.1326x over previous
import functools

import jax
import jax.numpy as jnp
from jax import lax
from jax.experimental import pallas as pl
from jax.experimental.pallas import tpu as pltpu
from jax.experimental.pallas import tpu_sc as plsc

BATCH = 16384
FIELDS = 26
OUT = 32
N_ROWS = 1000000
TOTAL = BATCH * FIELDS
NUM_CORES = 2
NUM_SUBCORES = 16
NW = NUM_CORES * NUM_SUBCORES
B_PER_W = BATCH // NW
PER_W = TOTAL // NW
B_CHUNK = 8
CHUNK = B_CHUNK * FIELDS
N_CHUNKS = PER_W // CHUNK
GROUPS = CHUNK // 16
DEPTH = 8
assert PER_W * NW == TOTAL and N_CHUNKS * CHUNK == PER_W
assert GROUPS * 16 == CHUNK and N_CHUNKS % DEPTH == 0 and DEPTH % 2 == 0


_T_BLK = 2048
_T_SUB = _T_BLK // 4
_T_GRID = (N_ROWS + _T_BLK - 1) // _T_BLK
_PACKED_ROWS = _T_GRID * _T_SUB


def _pack_body(x_ref, y_ref):
    x = x_ref[...]
    y_ref[...] = jnp.concatenate(
        [x[:, k * _T_SUB:(k + 1) * _T_SUB].T for k in range(4)], axis=1)


_pack_table = pl.pallas_call(
    _pack_body,
    grid=(_T_GRID,),
    in_specs=[pl.BlockSpec((OUT, _T_BLK), lambda i: (0, i))],
    out_specs=pl.BlockSpec((_T_SUB, 128), lambda i: (i, 0)),
    out_shape=jax.ShapeDtypeStruct((_PACKED_ROWS, 128), jnp.float32),
)


def _rsqrt(x):
    i = lax.bitcast_convert_type(x, jnp.int32)
    i = jnp.full((16,), 0x5F3759DF, jnp.int32) - (i >> 1)
    y = lax.bitcast_convert_type(i, jnp.float32)
    for _ in range(3):
        y = y * (1.5 - 0.5 * x * y * y)
    return y


_mesh = plsc.VectorSubcoreMesh(core_axis_name="c", subcore_axis_name="s")


@functools.partial(
    pl.kernel,
    out_type=jax.ShapeDtypeStruct((BATCH, FIELDS, OUT), jnp.float32),
    mesh=_mesh,
    scratch_types=[
        pltpu.VMEM((N_CHUNKS, CHUNK), jnp.int32),
        [pltpu.VMEM((CHUNK, OUT), jnp.float32) for _ in range(DEPTH)],
        [pltpu.VMEM((B_CHUNK, FIELDS, OUT), jnp.float32) for _ in range(2)],
        [pltpu.SemaphoreType.DMA for _ in range(DEPTH)],
        [pltpu.SemaphoreType.DMA for _ in range(2)],
    ],
    compiler_params=pltpu.CompilerParams(
        needs_layout_passes=False, use_tc_tiling_on_sc=False
    ),
)
def _gather_normalize(table_hbm, idx_hbm, out_hbm,
                      idx_v, gbufs, obufs, gsems, wsems):
    wid = lax.axis_index("s") * NUM_CORES + lax.axis_index("c")
    b_base = wid * B_PER_W

    lanes = lax.iota(jnp.int32, 16)

    def start_gather(g, p):
        pltpu.async_copy(table_hbm.at[idx_v.at[g]], gbufs[p], gsems[p])

    def wait_gather(g, p):
        pltpu.make_async_copy(table_hbm.at[idx_v.at[g]], gbufs[p],
                              gsems[p]).wait()

    def out_slice(g):
        return out_hbm.at[pl.ds(b_base + g * B_CHUNK, B_CHUNK)]

    def start_write(g, q):
        pltpu.async_copy(obufs[q], out_slice(g), wsems[q])

    def wait_write(g, q):
        pltpu.make_async_copy(obufs[q], out_slice(g), wsems[q]).wait()

    def normalize(p, q):
        src_v, dst_v = gbufs[p], obufs[q]

        def group_body(gr, c):
            row_ids = gr * 16 + lanes
            cols = [(lanes + j) & 31 for j in range(OUT)]
            elems = [plsc.load_gather(src_v, [row_ids, cols[j]])
                     for j in range(OUT)]
            acc = jnp.zeros((16,), jnp.float32)
            for e in elems:
                acc = acc + e * e
            inv = _rsqrt(acc)
            b_ids = row_ids // FIELDS
            f_ids = row_ids - b_ids * FIELDS
            for j, e in enumerate(elems):
                plsc.store_scatter(dst_v, [b_ids, f_ids, cols[j]], e * inv)
            return c

        lax.fori_loop(0, GROUPS, group_body, 0)

    pltpu.sync_copy(idx_hbm.at[wid], idx_v)
    for p in range(DEPTH):
        start_gather(p, p)

    def round_body(t, carry):
        for p in range(DEPTH):
            g = t * DEPTH + p
            q = p % 2
            wait_gather(g, p)

            @pl.when(g >= 2)
            def _():
                wait_write(g - 2, q)

            normalize(p, q)
            start_write(g, q)

            @pl.when(g + DEPTH < N_CHUNKS)
            def _():
                start_gather(g + DEPTH, p)

        return carry

    lax.fori_loop(0, N_CHUNKS // DEPTH, round_body, 0)
    wait_write(N_CHUNKS - 2, 0)
    wait_write(N_CHUNKS - 1, 1)


def kernel(src, categories_means, categories_logvars):
    del categories_logvars
    packed = _pack_table(categories_means.T)
    table = packed.reshape(_PACKED_ROWS * 4, OUT)
    t = src.astype(jnp.int32)
    idx = (t & ~(_T_BLK - 1)) + 4 * (t & (_T_SUB - 1)) + ((t >> 9) & 3)
    idx = idx.reshape(NW, N_CHUNKS, CHUNK)
    return _gather_normalize(table, idx)
